# baseline (speedup 1.0000x reference)
import functools

import jax
import jax.numpy as jnp
from jax import lax
from jax.experimental import pallas as pl
from jax.experimental.pallas import tpu as pltpu
from jax.experimental.pallas import tpu_sc as plsc

N = 10000
E = 320000
H = 128

NUM_TILES = 16
CHUNK = 50
EDGE_ROWS = E // CHUNK
ROWS_PER_TILE = EDGE_ROWS // NUM_TILES
N_PAD = 10240
OUT_ROWS_PER_TILE = N_PAD // NUM_TILES
ZCHUNK = 128
NBUF = 4
STEPS = ROWS_PER_TILE // NBUF


def _seg_body(h_hbm, c_hbm, src_hbm, dst_hbm, hout_hbm, cout_hbm,
              sbuf, dbuf, rows, zbuf, acc, sem_i, sem_g, sem_s):
    cid = lax.axis_index("c")
    sid = lax.axis_index("s")
    tile_row0 = sid * ROWS_PER_TILE

    zero = jnp.zeros((16,), jnp.float32)

    def _zrow(i, carry):
        for j in range(H // 16):
            zbuf[i, pl.ds(j * 16, 16)] = zero
        return carry

    lax.fori_loop(0, ZCHUNK, _zrow, 0)

    pltpu.sync_copy(src_hbm.at[pl.ds(tile_row0, NBUF)], sbuf.at[0])
    pltpu.sync_copy(dst_hbm.at[pl.ds(tile_row0, NBUF)], dbuf.at[0])

    for z in range(OUT_ROWS_PER_TILE // ZCHUNK):
        pltpu.sync_copy(
            zbuf, acc.at[pl.ds(sid * OUT_ROWS_PER_TILE + z * ZCHUNK, ZCHUNK)])
    plsc.subcore_barrier()

    def _run(table_hbm, out_hbm):
        for b in range(NBUF):
            pltpu.async_copy(table_hbm.at[pl.ds(b * CHUNK, CHUNK)],
                             rows.at[b], sem_g.at[b])

        def _turn(j, carry):
            p = j & 1
            pn = 1 - p
            @pl.when(j < STEPS - 1)
            def _():
                nxt = tile_row0 + (j + 1) * NBUF
                pltpu.async_copy(src_hbm.at[pl.ds(nxt, NBUF)],
                                 sbuf.at[pn], sem_i.at[0])
                pltpu.async_copy(dst_hbm.at[pl.ds(nxt, NBUF)],
                                 dbuf.at[pn], sem_i.at[1])

            for b in range(NBUF):
                pltpu.make_async_copy(table_hbm.at[sbuf.at[0, 0]],
                                      rows.at[b], sem_g.at[b]).wait()
                pltpu.async_copy(rows.at[b], acc.at[dbuf.at[p, b]],
                                 sem_s.at[b], add=True)

            @pl.when(j < STEPS - 1)
            def _():
                pltpu.make_async_copy(src_hbm.at[pl.ds(0, NBUF)],
                                      sbuf.at[0], sem_i.at[0]).wait()
                pltpu.make_async_copy(dst_hbm.at[pl.ds(0, NBUF)],
                                      dbuf.at[0], sem_i.at[1]).wait()
                for b in range(NBUF):
                    pltpu.make_async_copy(rows.at[b], acc.at[dbuf.at[0, 0]],
                                          sem_s.at[b]).wait()
                    off = ((j + 1) * NBUF + b) % 100
                    pltpu.async_copy(table_hbm.at[pl.ds(off * CHUNK, CHUNK)],
                                     rows.at[b], sem_g.at[b])
            return carry

        lax.fori_loop(0, STEPS, _turn, 0)
        for b in range(NBUF):
            pltpu.make_async_copy(rows.at[b], acc.at[dbuf.at[0, 0]],
                                  sem_s.at[b]).wait()
        plsc.subcore_barrier()
        for z in range(OUT_ROWS_PER_TILE // ZCHUNK):
            r0 = sid * OUT_ROWS_PER_TILE + z * ZCHUNK
            pltpu.sync_copy(acc.at[pl.ds(r0, ZCHUNK)],
                            out_hbm.at[pl.ds(r0, ZCHUNK)])

    @pl.when(cid == 0)
    def _():
        _run(h_hbm, hout_hbm)

    @pl.when(cid == 1)
    def _():
        _run(c_hbm, cout_hbm)


def _segment_sums(h, c, src2d, dst2d):
    full = jax.ShapeDtypeStruct((N_PAD, H), jnp.float32)
    kfn = functools.partial(
        pl.kernel,
        out_type=[full, full],
        mesh=plsc.VectorSubcoreMesh(core_axis_name="c", subcore_axis_name="s"),
        compiler_params=pltpu.CompilerParams(use_tc_tiling_on_sc=False),
        scratch_types=[
            pltpu.VMEM((2, NBUF, CHUNK), jnp.int32),
            pltpu.VMEM((2, NBUF, CHUNK), jnp.int32),
            pltpu.VMEM((NBUF, CHUNK, H), jnp.float32),
            pltpu.VMEM((ZCHUNK, H), jnp.float32),
            pltpu.VMEM_SHARED((N_PAD, H), jnp.float32),
            pltpu.SemaphoreType.DMA((2,)),
            pltpu.SemaphoreType.DMA((NBUF,)),
            pltpu.SemaphoreType.DMA((NBUF,)),
        ],
    )(_seg_body)
    return kfn(h, c, src2d, dst2d)


RB = 1000


def _dense_body(x_ref, hin_ref, cin_ref, w_ref, u_ref, b_ref,
                hout_ref, cout_ref):
    dn = (((1,), (1,)), ((), ()))
    g = (lax.dot_general(x_ref[...], w_ref[...], dn,
                         preferred_element_type=jnp.float32)
         + lax.dot_general(hin_ref[...], u_ref[...], dn,
                           preferred_element_type=jnp.float32)
         + b_ref[...])
    i = jax.nn.sigmoid(g[:, 0:H])
    o = jax.nn.sigmoid(g[:, H:2 * H])
    u = jnp.tanh(g[:, 2 * H:3 * H])
    f = jax.nn.sigmoid(g[:, 3 * H:4 * H])
    c_new = i * u + f * cin_ref[...]
    hout_ref[...] = o * jnp.tanh(c_new)
    cout_ref[...] = c_new


def _dense(x, h_in, c_in, W, U, b2d):
    grid = (N // RB,)
    row_spec = pl.BlockSpec((RB, H), lambda i: (i, 0))
    full_w = pl.BlockSpec((4 * H, H), lambda i: (0, 0))
    return pl.pallas_call(
        _dense_body,
        grid=grid,
        in_specs=[row_spec, row_spec, row_spec, full_w, full_w,
                  pl.BlockSpec((1, 4 * H), lambda i: (0, 0))],
        out_specs=[row_spec, row_spec],
        out_shape=[jax.ShapeDtypeStruct((N, H), jnp.float32),
                   jax.ShapeDtypeStruct((N, H), jnp.float32)],
    )(x, h_in, c_in, W, U, b2d)


def kernel(x, h, c, edge_index, W_iouf_w, W_iouf_b, U_iouf_w, U_iouf_b):
    src2d = edge_index[0].reshape(EDGE_ROWS, CHUNK)
    dst2d = edge_index[1].reshape(EDGE_ROWS, CHUNK)
    h_in, c_in = _segment_sums(h, c, src2d, dst2d)
    b2d = (W_iouf_b + U_iouf_b).reshape(1, 4 * H)
    return _dense(x, h_in, c_in, W_iouf_w, U_iouf_w, b2d)


# speedup vs baseline: 1.0410x; 1.0410x over previous
import functools

import jax
import jax.numpy as jnp
from jax import lax
from jax.experimental import pallas as pl
from jax.experimental.pallas import tpu as pltpu
from jax.experimental.pallas import tpu_sc as plsc

N = 10000
E = 320000
H = 128

NUM_TILES = 16
CHUNK = 50
EDGE_ROWS = E // CHUNK
ROWS_PER_TILE = EDGE_ROWS // NUM_TILES
N_PAD = 10240
OUT_ROWS_PER_TILE = N_PAD // NUM_TILES
ZCHUNK = 128
NBUF = 4
STEPS = ROWS_PER_TILE // NBUF


def _seg_body(h_hbm, c_hbm, ei_hbm, hout_hbm, cout_hbm,
              sbuf, dbuf, rows, zbuf, acc, sem_i, sem_g, sem_s):
    cid = lax.axis_index("c")
    sid = lax.axis_index("s")
    tile_row0 = sid * ROWS_PER_TILE

    zero = jnp.zeros((16,), jnp.float32)

    def _zrow(i, carry):
        for j in range(H // 16):
            zbuf[i, pl.ds(j * 16, 16)] = zero
        return carry

    lax.fori_loop(0, ZCHUNK, _zrow, 0)

    pltpu.sync_copy(ei_hbm.at[0, pl.ds(tile_row0, NBUF)], sbuf.at[0])
    pltpu.sync_copy(ei_hbm.at[1, pl.ds(tile_row0, NBUF)], dbuf.at[0])

    def _run(table_hbm, out_hbm):
        for b in range(NBUF):
            pltpu.async_copy(table_hbm.at[sbuf.at[0, b]],
                             rows.at[b], sem_g.at[b])

        for z in range(OUT_ROWS_PER_TILE // ZCHUNK):
            pltpu.sync_copy(
                zbuf,
                acc.at[pl.ds(sid * OUT_ROWS_PER_TILE + z * ZCHUNK, ZCHUNK)])
        plsc.subcore_barrier()

        def _turn(j, carry):
            p = j & 1
            pn = 1 - p
            @pl.when(j < STEPS - 1)
            def _():
                nxt = tile_row0 + (j + 1) * NBUF
                pltpu.async_copy(ei_hbm.at[0, pl.ds(nxt, NBUF)],
                                 sbuf.at[pn], sem_i.at[0])
                pltpu.async_copy(ei_hbm.at[1, pl.ds(nxt, NBUF)],
                                 dbuf.at[pn], sem_i.at[1])

            for b in range(NBUF):
                pltpu.make_async_copy(table_hbm.at[sbuf.at[0, 0]],
                                      rows.at[b], sem_g.at[b]).wait()
                pltpu.async_copy(rows.at[b], acc.at[dbuf.at[p, b]],
                                 sem_s.at[b], add=True)

            @pl.when(j < STEPS - 1)
            def _():
                pltpu.make_async_copy(ei_hbm.at[0, pl.ds(0, NBUF)],
                                      sbuf.at[0], sem_i.at[0]).wait()
                pltpu.make_async_copy(ei_hbm.at[1, pl.ds(0, NBUF)],
                                      dbuf.at[0], sem_i.at[1]).wait()
                for b in range(NBUF):
                    pltpu.make_async_copy(rows.at[b], acc.at[dbuf.at[0, 0]],
                                          sem_s.at[b]).wait()
                    pltpu.async_copy(table_hbm.at[sbuf.at[pn, b]],
                                     rows.at[b], sem_g.at[b])
            return carry

        lax.fori_loop(0, STEPS, _turn, 0)
        for b in range(NBUF):
            pltpu.make_async_copy(rows.at[b], acc.at[dbuf.at[0, 0]],
                                  sem_s.at[b]).wait()
        plsc.subcore_barrier()
        for z in range(OUT_ROWS_PER_TILE // ZCHUNK):
            r0 = sid * OUT_ROWS_PER_TILE + z * ZCHUNK
            pltpu.sync_copy(acc.at[pl.ds(r0, ZCHUNK)],
                            out_hbm.at[pl.ds(r0, ZCHUNK)])

    @pl.when(cid == 0)
    def _():
        _run(h_hbm, hout_hbm)

    @pl.when(cid == 1)
    def _():
        _run(c_hbm, cout_hbm)


def _segment_sums(h, c, ei3):
    full = jax.ShapeDtypeStruct((N_PAD, H), jnp.float32)
    kfn = functools.partial(
        pl.kernel,
        out_type=[full, full],
        mesh=plsc.VectorSubcoreMesh(core_axis_name="c", subcore_axis_name="s"),
        compiler_params=pltpu.CompilerParams(use_tc_tiling_on_sc=False),
        scratch_types=[
            pltpu.VMEM((2, NBUF, CHUNK), jnp.int32),
            pltpu.VMEM((2, NBUF, CHUNK), jnp.int32),
            pltpu.VMEM((NBUF, CHUNK, H), jnp.float32),
            pltpu.VMEM((ZCHUNK, H), jnp.float32),
            pltpu.VMEM_SHARED((N_PAD, H), jnp.float32),
            pltpu.SemaphoreType.DMA((2,)),
            pltpu.SemaphoreType.DMA((NBUF,)),
            pltpu.SemaphoreType.DMA((NBUF,)),
        ],
    )(_seg_body)
    return kfn(h, c, ei3)


RB = 1000


def _dense_body(x_ref, hin_ref, cin_ref, w_ref, u_ref, b_ref,
                hout_ref, cout_ref):
    dn = (((1,), (1,)), ((), ()))
    g = (lax.dot_general(x_ref[...], w_ref[...], dn,
                         preferred_element_type=jnp.float32)
         + lax.dot_general(hin_ref[...], u_ref[...], dn,
                           preferred_element_type=jnp.float32)
         + b_ref[...])
    i = jax.nn.sigmoid(g[:, 0:H])
    o = jax.nn.sigmoid(g[:, H:2 * H])
    u = jnp.tanh(g[:, 2 * H:3 * H])
    f = jax.nn.sigmoid(g[:, 3 * H:4 * H])
    c_new = i * u + f * cin_ref[...]
    hout_ref[...] = o * jnp.tanh(c_new)
    cout_ref[...] = c_new


def _dense(x, h_in, c_in, W, U, b2d):
    grid = (N // RB,)
    row_spec = pl.BlockSpec((RB, H), lambda i: (i, 0))
    full_w = pl.BlockSpec((4 * H, H), lambda i: (0, 0))
    return pl.pallas_call(
        _dense_body,
        grid=grid,
        in_specs=[row_spec, row_spec, row_spec, full_w, full_w,
                  pl.BlockSpec((1, 4 * H), lambda i: (0, 0))],
        out_specs=[row_spec, row_spec],
        out_shape=[jax.ShapeDtypeStruct((N, H), jnp.float32),
                   jax.ShapeDtypeStruct((N, H), jnp.float32)],
    )(x, h_in, c_in, W, U, b2d)


def kernel(x, h, c, edge_index, W_iouf_w, W_iouf_b, U_iouf_w, U_iouf_b):
    ei3 = edge_index.reshape(2, EDGE_ROWS, CHUNK)
    h_in, c_in = _segment_sums(h, c, ei3)
    b2d = (W_iouf_b + U_iouf_b).reshape(1, 4 * H)
    return _dense(x, h_in, c_in, W_iouf_w, U_iouf_w, b2d)


# speedup vs baseline: 1.0499x; 1.0086x over previous
import functools

import jax
import jax.numpy as jnp
from jax import lax
from jax.experimental import pallas as pl
from jax.experimental.pallas import tpu as pltpu
from jax.experimental.pallas import tpu_sc as plsc

N = 10000
E = 320000
H = 128

NUM_TILES = 16
CHUNK = 50
EDGE_ROWS = E // CHUNK
ROWS_PER_TILE = EDGE_ROWS // NUM_TILES
N_PAD = 10240
OUT_ROWS_PER_TILE = N_PAD // NUM_TILES
ZCHUNK = 128
NBUF = 4
STEPS = ROWS_PER_TILE // NBUF


def _seg_body(h_hbm, c_hbm, ei_hbm, hout_hbm, cout_hbm,
              sbuf, dbuf, rows, zbuf, acc, sem_i, sem_g, sem_s):
    cid = lax.axis_index("c")
    sid = lax.axis_index("s")
    tile_row0 = sid * ROWS_PER_TILE

    zero = jnp.zeros((16,), jnp.float32)

    def _zrow(i, carry):
        for j in range(H // 16):
            zbuf[i, pl.ds(j * 16, 16)] = zero
        return carry

    lax.fori_loop(0, ZCHUNK, _zrow, 0)

    pltpu.sync_copy(ei_hbm.at[0, pl.ds(tile_row0, NBUF)], sbuf.at[0])
    pltpu.sync_copy(ei_hbm.at[1, pl.ds(tile_row0, NBUF)], dbuf.at[0])

    def _run(table_hbm, out_hbm):
        for b in range(NBUF):
            pltpu.async_copy(table_hbm.at[sbuf.at[0, b]],
                             rows.at[b], sem_g.at[b])

        for z in range(OUT_ROWS_PER_TILE // ZCHUNK):
            pltpu.sync_copy(
                zbuf,
                acc.at[pl.ds(sid * OUT_ROWS_PER_TILE + z * ZCHUNK, ZCHUNK)])
        plsc.subcore_barrier()

        def _turn(j, carry):
            p = j & 1
            pn = 1 - p
            @pl.when(j < STEPS - 1)
            def _():
                nxt = tile_row0 + (j + 1) * NBUF
                pltpu.async_copy(ei_hbm.at[0, pl.ds(nxt, NBUF)],
                                 sbuf.at[pn], sem_i.at[0])
                pltpu.async_copy(ei_hbm.at[1, pl.ds(nxt, NBUF)],
                                 dbuf.at[pn], sem_i.at[1])

            for b in range(NBUF):
                pltpu.make_async_copy(table_hbm.at[sbuf.at[0, 0]],
                                      rows.at[b], sem_g.at[b]).wait()
                pltpu.async_copy(rows.at[b], acc.at[dbuf.at[p, b]],
                                 sem_s.at[b], add=True)

            @pl.when(j < STEPS - 1)
            def _():
                pltpu.make_async_copy(ei_hbm.at[0, pl.ds(0, NBUF)],
                                      sbuf.at[0], sem_i.at[0]).wait()
                pltpu.make_async_copy(ei_hbm.at[1, pl.ds(0, NBUF)],
                                      dbuf.at[0], sem_i.at[1]).wait()
                for b in range(NBUF):
                    pltpu.make_async_copy(rows.at[b], acc.at[dbuf.at[0, 0]],
                                          sem_s.at[b]).wait()
                    pltpu.async_copy(table_hbm.at[sbuf.at[pn, b]],
                                     rows.at[b], sem_g.at[b])
            return carry

        lax.fori_loop(0, STEPS, _turn, 0)
        for b in range(NBUF):
            pltpu.make_async_copy(rows.at[b], acc.at[dbuf.at[0, 0]],
                                  sem_s.at[b]).wait()
        plsc.subcore_barrier()
        for z in range(OUT_ROWS_PER_TILE // ZCHUNK):
            r0 = sid * OUT_ROWS_PER_TILE + z * ZCHUNK
            pltpu.sync_copy(acc.at[pl.ds(r0, ZCHUNK)],
                            out_hbm.at[pl.ds(r0, ZCHUNK)])

    @pl.when(cid == 0)
    def _():
        _run(h_hbm, hout_hbm)

    @pl.when(cid == 1)
    def _():
        _run(c_hbm, cout_hbm)


def _segment_sums(h, c, ei3):
    full = jax.ShapeDtypeStruct((N_PAD, H), jnp.float32)
    kfn = functools.partial(
        pl.kernel,
        out_type=[full, full],
        mesh=plsc.VectorSubcoreMesh(core_axis_name="c", subcore_axis_name="s"),
        compiler_params=pltpu.CompilerParams(use_tc_tiling_on_sc=False),
        scratch_types=[
            pltpu.VMEM((2, NBUF, CHUNK), jnp.int32),
            pltpu.VMEM((2, NBUF, CHUNK), jnp.int32),
            pltpu.VMEM((NBUF, CHUNK, H), jnp.float32),
            pltpu.VMEM((ZCHUNK, H), jnp.float32),
            pltpu.VMEM_SHARED((N_PAD, H), jnp.float32),
            pltpu.SemaphoreType.DMA((2,)),
            pltpu.SemaphoreType.DMA((NBUF,)),
            pltpu.SemaphoreType.DMA((NBUF,)),
        ],
    )(_seg_body)
    return kfn(h, c, ei3)


RB = 2000


def _dense_body(x_ref, hin_ref, cin_ref, w_ref, u_ref, b1_ref, b2_ref,
                hout_ref, cout_ref):
    dn = (((1,), (1,)), ((), ()))
    g = (lax.dot_general(x_ref[...], w_ref[...], dn,
                         preferred_element_type=jnp.float32)
         + lax.dot_general(hin_ref[...], u_ref[...], dn,
                           preferred_element_type=jnp.float32)
         + (b1_ref[...] + b2_ref[...]))
    i = jax.nn.sigmoid(g[:, 0:H])
    o = jax.nn.sigmoid(g[:, H:2 * H])
    u = jnp.tanh(g[:, 2 * H:3 * H])
    f = jax.nn.sigmoid(g[:, 3 * H:4 * H])
    c_new = i * u + f * cin_ref[...]
    hout_ref[...] = o * jnp.tanh(c_new)
    cout_ref[...] = c_new


def _dense(x, h_in, c_in, W, U, b1, b2):
    grid = (N // RB,)
    row_spec = pl.BlockSpec((RB, H), lambda i: (i, 0))
    full_w = pl.BlockSpec((4 * H, H), lambda i: (0, 0))
    bias_spec = pl.BlockSpec((1, 4 * H), lambda i: (0, 0))
    return pl.pallas_call(
        _dense_body,
        grid=grid,
        in_specs=[row_spec, row_spec, row_spec, full_w, full_w,
                  bias_spec, bias_spec],
        out_specs=[row_spec, row_spec],
        out_shape=[jax.ShapeDtypeStruct((N, H), jnp.float32),
                   jax.ShapeDtypeStruct((N, H), jnp.float32)],
    )(x, h_in, c_in, W, U, b1, b2)


def kernel(x, h, c, edge_index, W_iouf_w, W_iouf_b, U_iouf_w, U_iouf_b):
    ei3 = edge_index.reshape(2, EDGE_ROWS, CHUNK)
    h_in, c_in = _segment_sums(h, c, ei3)
    return _dense(x, h_in, c_in, W_iouf_w, U_iouf_w,
                  W_iouf_b.reshape(1, 4 * H), U_iouf_b.reshape(1, 4 * H))


# speedup vs baseline: 1.0926x; 1.0406x over previous
import functools

import jax
import jax.numpy as jnp
from jax import lax
from jax.experimental import pallas as pl
from jax.experimental.pallas import tpu as pltpu
from jax.experimental.pallas import tpu_sc as plsc

N = 10000
E = 320000
H = 128

NUM_TILES = 16
CHUNK = 40
EDGE_ROWS = E // CHUNK
ROWS_PER_TILE = EDGE_ROWS // NUM_TILES
N_PAD = 10240
OUT_ROWS_PER_TILE = N_PAD // NUM_TILES
ZCHUNK = 128
NBUF = 4
STEPS = ROWS_PER_TILE // NBUF


def _seg_body(h_hbm, c_hbm, ei_hbm, hout_hbm, cout_hbm,
              sbuf, dbuf, rows, acc, sem_i, sem_g, sem_s):
    cid = lax.axis_index("c")
    sid = lax.axis_index("s")
    tile_row0 = sid * ROWS_PER_TILE

    zero = jnp.zeros((16,), jnp.float32)

    def _zrow(i, carry):
        for j in range(H // 16):
            rows[NBUF, i, pl.ds(j * 16, 16)] = zero
        return carry

    lax.fori_loop(0, CHUNK, _zrow, 0)

    pltpu.sync_copy(ei_hbm.at[0, pl.ds(tile_row0, NBUF)], sbuf.at[0])
    pltpu.sync_copy(ei_hbm.at[1, pl.ds(tile_row0, NBUF)], dbuf.at[0])

    def _run(table_hbm, out_hbm):
        for b in range(NBUF):
            pltpu.async_copy(table_hbm.at[sbuf.at[0, b]],
                             rows.at[b], sem_g.at[b])

        r0 = sid * OUT_ROWS_PER_TILE
        for z in range(OUT_ROWS_PER_TILE // CHUNK):
            pltpu.sync_copy(rows.at[NBUF], acc.at[pl.ds(r0 + z * CHUNK, CHUNK)])
        rem = OUT_ROWS_PER_TILE % CHUNK
        if rem:
            pltpu.sync_copy(rows.at[NBUF, pl.ds(0, rem)],
                            acc.at[pl.ds(r0 + OUT_ROWS_PER_TILE - rem, rem)])
        plsc.subcore_barrier()

        def _one_turn(j, u, tail):
            un = 1 - u
            def _drain_prev():
                for b in range(NBUF):
                    pltpu.make_async_copy(
                        rows.at[un * NBUF + b], acc.at[dbuf.at[0, 0]],
                        sem_s.at[un * NBUF + b]).wait()

            if tail:
                _drain_prev()
            else:
                @pl.when(j > 0)
                def _():
                    _drain_prev()

            if not tail:
                @pl.when(j < STEPS - 1)
                def _():
                    nxt = tile_row0 + (j + 1) * NBUF
                    pltpu.async_copy(ei_hbm.at[0, pl.ds(nxt, NBUF)],
                                     sbuf.at[un], sem_i.at[0])
                    pltpu.async_copy(ei_hbm.at[1, pl.ds(nxt, NBUF)],
                                     dbuf.at[un], sem_i.at[1])

            for b in range(NBUF):
                pltpu.make_async_copy(table_hbm.at[sbuf.at[0, 0]],
                                      rows.at[u * NBUF + b],
                                      sem_g.at[u * NBUF + b]).wait()
                pltpu.async_copy(rows.at[u * NBUF + b],
                                 acc.at[dbuf.at[u, b]],
                                 sem_s.at[u * NBUF + b], add=True)

            if not tail:
                @pl.when(j < STEPS - 1)
                def _():
                    pltpu.make_async_copy(ei_hbm.at[0, pl.ds(0, NBUF)],
                                          sbuf.at[0], sem_i.at[0]).wait()
                    pltpu.make_async_copy(ei_hbm.at[1, pl.ds(0, NBUF)],
                                          dbuf.at[0], sem_i.at[1]).wait()
                    for b in range(NBUF):
                        pltpu.async_copy(table_hbm.at[sbuf.at[un, b]],
                                         rows.at[un * NBUF + b],
                                         sem_g.at[un * NBUF + b])

        def _turn2(t, carry):
            _one_turn(2 * t, 0, False)
            _one_turn(2 * t + 1, 1, False)
            return carry

        lax.fori_loop(0, STEPS // 2, _turn2, 0)
        _one_turn(STEPS - 1, 0, True)
        for b in range(NBUF):
            pltpu.make_async_copy(rows.at[b], acc.at[dbuf.at[0, 0]],
                                  sem_s.at[b]).wait()
        plsc.subcore_barrier()
        for z in range(OUT_ROWS_PER_TILE // ZCHUNK):
            w0 = sid * OUT_ROWS_PER_TILE + z * ZCHUNK
            pltpu.sync_copy(acc.at[pl.ds(w0, ZCHUNK)],
                            out_hbm.at[pl.ds(w0, ZCHUNK)])

    @pl.when(cid == 0)
    def _():
        _run(h_hbm, hout_hbm)

    @pl.when(cid == 1)
    def _():
        _run(c_hbm, cout_hbm)


def _segment_sums(h, c, ei3):
    full = jax.ShapeDtypeStruct((N_PAD, H), jnp.float32)
    kfn = functools.partial(
        pl.kernel,
        out_type=[full, full],
        mesh=plsc.VectorSubcoreMesh(core_axis_name="c", subcore_axis_name="s"),
        compiler_params=pltpu.CompilerParams(use_tc_tiling_on_sc=False),
        scratch_types=[
            pltpu.VMEM((2, NBUF, CHUNK), jnp.int32),
            pltpu.VMEM((2, NBUF, CHUNK), jnp.int32),
            pltpu.VMEM((2 * NBUF, CHUNK, H), jnp.float32),
            pltpu.VMEM_SHARED((N_PAD, H), jnp.float32),
            pltpu.SemaphoreType.DMA((2,)),
            pltpu.SemaphoreType.DMA((2 * NBUF,)),
            pltpu.SemaphoreType.DMA((2 * NBUF,)),
        ],
    )(_seg_body)
    return kfn(h, c, ei3)


RB = 2000


def _dense_body(x_ref, hin_ref, cin_ref, w_ref, u_ref, b1_ref, b2_ref,
                hout_ref, cout_ref):
    dn = (((1,), (1,)), ((), ()))
    g = (lax.dot_general(x_ref[...], w_ref[...], dn,
                         preferred_element_type=jnp.float32)
         + lax.dot_general(hin_ref[...], u_ref[...], dn,
                           preferred_element_type=jnp.float32)
         + (b1_ref[...] + b2_ref[...]))
    i = jax.nn.sigmoid(g[:, 0:H])
    o = jax.nn.sigmoid(g[:, H:2 * H])
    u = jnp.tanh(g[:, 2 * H:3 * H])
    f = jax.nn.sigmoid(g[:, 3 * H:4 * H])
    c_new = i * u + f * cin_ref[...]
    hout_ref[...] = o * jnp.tanh(c_new)
    cout_ref[...] = c_new


def _dense(x, h_in, c_in, W, U, b1, b2):
    grid = (N // RB,)
    row_spec = pl.BlockSpec((RB, H), lambda i: (i, 0))
    full_w = pl.BlockSpec((4 * H, H), lambda i: (0, 0))
    bias_spec = pl.BlockSpec((1, 4 * H), lambda i: (0, 0))
    return pl.pallas_call(
        _dense_body,
        grid=grid,
        in_specs=[row_spec, row_spec, row_spec, full_w, full_w,
                  bias_spec, bias_spec],
        out_specs=[row_spec, row_spec],
        out_shape=[jax.ShapeDtypeStruct((N, H), jnp.float32),
                   jax.ShapeDtypeStruct((N, H), jnp.float32)],
    )(x, h_in, c_in, W, U, b1, b2)


def kernel(x, h, c, edge_index, W_iouf_w, W_iouf_b, U_iouf_w, U_iouf_b):
    ei3 = edge_index.reshape(2, EDGE_ROWS, CHUNK)
    h_in, c_in = _segment_sums(h, c, ei3)
    return _dense(x, h_in, c_in, W_iouf_w, U_iouf_w,
                  W_iouf_b.reshape(1, 4 * H), U_iouf_b.reshape(1, 4 * H))
